# baseline (speedup 1.0000x reference)
import functools

import jax
import jax.numpy as jnp
from jax import lax
from jax.experimental import pallas as pl
from jax.experimental.pallas import tpu as pltpu
from jax.experimental.pallas import tpu_sc as plsc

N = 10000
E = 320000
D = 128
T = 4
STEPS = 8
B = 10
NPG = 1000

NC = 2
NS = 16
NW = NC * NS
EPW = E // NW
K = 128
NCH = (EPW + K - 1) // K
EPP = NCH * K
TRASH = 10200
NP_ = 10240
RPS = NP_ // NS


@functools.lru_cache(maxsize=None)
def _get_sc_aggregate():
    mesh = plsc.VectorSubcoreMesh(core_axis_name="c", subcore_axis_name="s")

    @functools.partial(
        pl.kernel,
        mesh=mesh,
        out_type=jax.ShapeDtypeStruct((NC, NP_, D), jnp.float32),
        scratch_types=[
            pltpu.VMEM((NCH, K), jnp.int32),
            pltpu.VMEM((NCH, K), jnp.int32),
            pltpu.VMEM((K, D), jnp.float32),
            pltpu.VMEM_SHARED((NP_, D), jnp.float32),
            pltpu.SemaphoreType.DMA,
        ],
    )
    def _sc_aggregate(table, gidxr, dstr, zeros, out, gidx_v, dst_v, rows, accum, sem):
        s = lax.axis_index("s")
        c = lax.axis_index("c")
        wid = s * NC + c

        pltpu.sync_copy(gidxr.at[wid], gidx_v)
        pltpu.sync_copy(dstr.at[wid], dst_v)
        pltpu.sync_copy(zeros.at[pl.ds(s * RPS, RPS)], accum.at[pl.ds(s * RPS, RPS)])
        plsc.subcore_barrier()

        def body(j, carry):
            pltpu.async_copy(table.at[gidx_v.at[j]], rows, sem).wait()
            pltpu.sync_copy(rows, accum.at[dst_v.at[j]], add=True)
            return carry

        lax.fori_loop(0, NCH, body, 0)

        plsc.subcore_barrier()
        pltpu.sync_copy(accum.at[pl.ds(s * RPS, RPS)],
                        out.at[c, pl.ds(s * RPS, RPS)])

    return _sc_aggregate


_BN = 2000


def _transform_body(h_ref, w_ref, b_ref, out_ref):
    h = h_ref[...]
    w = w_ref[0]
    out_ref[0] = lax.dot_general(h, w, (((1,), (1,)), ((), ())),
                                 preferred_element_type=jnp.float32) + b_ref[0]


def _tc_transform(h, W_et, b_et):
    return pl.pallas_call(
        _transform_body,
        grid=(N // _BN, T),
        in_specs=[
            pl.BlockSpec((_BN, D), lambda i, t: (i, 0)),
            pl.BlockSpec((1, D, D), lambda i, t: (t, 0, 0)),
            pl.BlockSpec((1, 1, D), lambda i, t: (t, 0, 0)),
        ],
        out_specs=pl.BlockSpec((1, _BN, D), lambda i, t: (t, i, 0)),
        out_shape=jax.ShapeDtypeStruct((T, N, D), jnp.float32),
    )(h, W_et, b_et)


def _gru_body(p0_ref, p1_ref, h_ref, wih_ref, whh_ref, bih_ref, bhh_ref, out_ref):
    a = p0_ref[...] + p1_ref[...]
    h = h_ref[...]
    gi = lax.dot_general(a, wih_ref[...], (((1,), (1,)), ((), ())),
                         preferred_element_type=jnp.float32) + bih_ref[...]
    gh = lax.dot_general(h, whh_ref[...], (((1,), (1,)), ((), ())),
                         preferred_element_type=jnp.float32) + bhh_ref[...]
    r = jax.nn.sigmoid(gi[:, :D] + gh[:, :D])
    z = jax.nn.sigmoid(gi[:, D:2 * D] + gh[:, D:2 * D])
    n = jnp.tanh(gi[:, 2 * D:] + r * gh[:, 2 * D:])
    out_ref[...] = (1.0 - z) * n + z * h


def _tc_gru(p0, p1, h, W_ih, W_hh, b_ih, b_hh):
    return pl.pallas_call(
        _gru_body,
        grid=(N // _BN,),
        in_specs=[
            pl.BlockSpec((_BN, D), lambda i: (i, 0)),
            pl.BlockSpec((_BN, D), lambda i: (i, 0)),
            pl.BlockSpec((_BN, D), lambda i: (i, 0)),
            pl.BlockSpec((3 * D, D), lambda i: (0, 0)),
            pl.BlockSpec((3 * D, D), lambda i: (0, 0)),
            pl.BlockSpec((1, 3 * D), lambda i: (0, 0)),
            pl.BlockSpec((1, 3 * D), lambda i: (0, 0)),
        ],
        out_specs=pl.BlockSpec((_BN, D), lambda i: (i, 0)),
        out_shape=jax.ShapeDtypeStruct((N, D), jnp.float32),
    )(p0, p1, h, W_ih, W_hh, b_ih, b_hh)


def _pool_body(h_ref, w_ref, b_ref, out_ref):
    s = jnp.sum(h_ref[...], axis=0, keepdims=True)
    logit = jnp.sum(s * w_ref[...], axis=1, keepdims=True)
    out_ref[0] = jax.nn.sigmoid(logit + b_ref[...])


def _tc_pool(h, W_cls, b_cls):
    return pl.pallas_call(
        _pool_body,
        grid=(B,),
        in_specs=[
            pl.BlockSpec((NPG, D), lambda i: (i, 0)),
            pl.BlockSpec((1, D), lambda i: (0, 0)),
            pl.BlockSpec((1, D), lambda i: (0, 0)),
        ],
        out_specs=pl.BlockSpec((1, 1, D), lambda i: (i, 0, 0)),
        out_shape=jax.ShapeDtypeStruct((B, 1, D), jnp.float32),
    )(h, W_cls, b_cls)


def kernel(features, edge_index, edge_types, W_et, b_et, W_ih, W_hh, b_ih, b_hh, W_cls, b_cls):
    src = edge_index[0]
    dst = edge_index[1]
    pad = EPP - EPW
    gidx = jnp.pad((edge_types * N + src).reshape(NW, EPW),
                   ((0, 0), (0, pad))).reshape(NW, NCH, K)
    dst3 = jnp.pad(dst.reshape(NW, EPW), ((0, 0), (0, pad)),
                   constant_values=TRASH).reshape(NW, NCH, K)
    zeros = jnp.zeros((NP_, D), jnp.float32)
    b_ih2 = b_ih.reshape(1, 3 * D)
    b_hh2 = b_hh.reshape(1, 3 * D)
    b_cls2 = jnp.broadcast_to(b_cls.reshape(1, 1), (1, D))

    h = features
    for _ in range(STEPS):
        table = _tc_transform(h, W_et, b_et.reshape(T, 1, D)).reshape(T * N, D)
        parts = _get_sc_aggregate()(table, gidx, dst3, zeros)
        h = _tc_gru(parts[0, :N], parts[1, :N], h, W_ih, W_hh, b_ih2, b_hh2)
    out = _tc_pool(h, W_cls, b_cls2)
    return out[:, 0, 0]


# speedup vs baseline: 1.0863x; 1.0863x over previous
import functools

import jax
import jax.numpy as jnp
from jax import lax
from jax.experimental import pallas as pl
from jax.experimental.pallas import tpu as pltpu
from jax.experimental.pallas import tpu_sc as plsc

N = 10000
E = 320000
D = 128
T = 4
STEPS = 8
B = 10
NPG = 1000

NC = 2
NS = 16
NW = NC * NS
EPW = E // NW
K = 64
NCH = (EPW + K - 1) // K
EPP = NCH * K
TRASH = 10200
NP_ = 10240
RPS = NP_ // NS


@functools.lru_cache(maxsize=None)
def _get_sc_aggregate():
    mesh = plsc.VectorSubcoreMesh(core_axis_name="c", subcore_axis_name="s")

    @functools.partial(
        pl.kernel,
        mesh=mesh,
        out_type=jax.ShapeDtypeStruct((NC, NP_, D), jnp.float32),
        scratch_types=[
            pltpu.VMEM((NCH, K), jnp.int32),
            pltpu.VMEM((NCH, K), jnp.int32),
            pltpu.VMEM((K, D), jnp.float32),
            pltpu.VMEM_SHARED((NP_, D), jnp.float32),
            pltpu.SemaphoreType.DMA,
        ],
    )
    def _sc_aggregate(table, gidxr, dstr, zeros, out, gidx_v, dst_v, rows, accum, sem):
        s = lax.axis_index("s")
        c = lax.axis_index("c")
        wid = s * NC + c

        pltpu.sync_copy(gidxr.at[wid], gidx_v)
        pltpu.sync_copy(dstr.at[wid], dst_v)
        pltpu.sync_copy(zeros.at[pl.ds(s * RPS, RPS)], accum.at[pl.ds(s * RPS, RPS)])
        plsc.subcore_barrier()

        def body(j, carry):
            pltpu.async_copy(table.at[gidx_v.at[j]], rows, sem).wait()
            pltpu.sync_copy(rows, accum.at[dst_v.at[j]], add=True)
            return carry

        lax.fori_loop(0, NCH, body, 0)

        plsc.subcore_barrier()
        pltpu.sync_copy(accum.at[pl.ds(s * RPS, RPS)],
                        out.at[c, pl.ds(s * RPS, RPS)])

    return _sc_aggregate


_BN = 2000


def _transform_body(h_ref, w_ref, b_ref, out_ref):
    h = h_ref[...]
    w = w_ref[0]
    out_ref[0] = lax.dot_general(h, w, (((1,), (1,)), ((), ())),
                                 preferred_element_type=jnp.float32) + b_ref[0]


def _tc_transform(h, W_et, b_et):
    return pl.pallas_call(
        _transform_body,
        grid=(N // _BN, T),
        in_specs=[
            pl.BlockSpec((_BN, D), lambda i, t: (i, 0)),
            pl.BlockSpec((1, D, D), lambda i, t: (t, 0, 0)),
            pl.BlockSpec((1, 1, D), lambda i, t: (t, 0, 0)),
        ],
        out_specs=pl.BlockSpec((1, _BN, D), lambda i, t: (t, i, 0)),
        out_shape=jax.ShapeDtypeStruct((T, N, D), jnp.float32),
    )(h, W_et, b_et)


def _gru_body(p0_ref, p1_ref, h_ref, wih_ref, whh_ref, bih_ref, bhh_ref, out_ref):
    a = p0_ref[...] + p1_ref[...]
    h = h_ref[...]
    gi = lax.dot_general(a, wih_ref[...], (((1,), (1,)), ((), ())),
                         preferred_element_type=jnp.float32) + bih_ref[...]
    gh = lax.dot_general(h, whh_ref[...], (((1,), (1,)), ((), ())),
                         preferred_element_type=jnp.float32) + bhh_ref[...]
    r = jax.nn.sigmoid(gi[:, :D] + gh[:, :D])
    z = jax.nn.sigmoid(gi[:, D:2 * D] + gh[:, D:2 * D])
    n = jnp.tanh(gi[:, 2 * D:] + r * gh[:, 2 * D:])
    out_ref[...] = (1.0 - z) * n + z * h


def _tc_gru(p0, p1, h, W_ih, W_hh, b_ih, b_hh):
    return pl.pallas_call(
        _gru_body,
        grid=(N // _BN,),
        in_specs=[
            pl.BlockSpec((_BN, D), lambda i: (i, 0)),
            pl.BlockSpec((_BN, D), lambda i: (i, 0)),
            pl.BlockSpec((_BN, D), lambda i: (i, 0)),
            pl.BlockSpec((3 * D, D), lambda i: (0, 0)),
            pl.BlockSpec((3 * D, D), lambda i: (0, 0)),
            pl.BlockSpec((1, 3 * D), lambda i: (0, 0)),
            pl.BlockSpec((1, 3 * D), lambda i: (0, 0)),
        ],
        out_specs=pl.BlockSpec((_BN, D), lambda i: (i, 0)),
        out_shape=jax.ShapeDtypeStruct((N, D), jnp.float32),
    )(p0, p1, h, W_ih, W_hh, b_ih, b_hh)


def _pool_body(h_ref, w_ref, b_ref, out_ref):
    s = jnp.sum(h_ref[...], axis=0, keepdims=True)
    logit = jnp.sum(s * w_ref[...], axis=1, keepdims=True)
    out_ref[0] = jax.nn.sigmoid(logit + b_ref[...])


def _tc_pool(h, W_cls, b_cls):
    return pl.pallas_call(
        _pool_body,
        grid=(B,),
        in_specs=[
            pl.BlockSpec((NPG, D), lambda i: (i, 0)),
            pl.BlockSpec((1, D), lambda i: (0, 0)),
            pl.BlockSpec((1, D), lambda i: (0, 0)),
        ],
        out_specs=pl.BlockSpec((1, 1, D), lambda i: (i, 0, 0)),
        out_shape=jax.ShapeDtypeStruct((B, 1, D), jnp.float32),
    )(h, W_cls, b_cls)


def kernel(features, edge_index, edge_types, W_et, b_et, W_ih, W_hh, b_ih, b_hh, W_cls, b_cls):
    src = edge_index[0]
    dst = edge_index[1]
    pad = EPP - EPW
    gidx = jnp.pad((edge_types * N + src).reshape(NW, EPW),
                   ((0, 0), (0, pad))).reshape(NW, NCH, K)
    dst3 = jnp.pad(dst.reshape(NW, EPW), ((0, 0), (0, pad)),
                   constant_values=TRASH).reshape(NW, NCH, K)
    zeros = jnp.zeros((NP_, D), jnp.float32)
    b_ih2 = b_ih.reshape(1, 3 * D)
    b_hh2 = b_hh.reshape(1, 3 * D)
    b_cls2 = jnp.broadcast_to(b_cls.reshape(1, 1), (1, D))

    h = features
    for _ in range(STEPS):
        table = _tc_transform(h, W_et, b_et.reshape(T, 1, D)).reshape(T * N, D)
        parts = _get_sc_aggregate()(table, gidx, dst3, zeros)
        h = _tc_gru(parts[0, :N], parts[1, :N], h, W_ih, W_hh, b_ih2, b_hh2)
    out = _tc_pool(h, W_cls, b_cls2)
    return out[:, 0, 0]


# speedup vs baseline: 1.5001x; 1.3809x over previous
import functools

import jax
import jax.numpy as jnp
from jax import lax
from jax.experimental import pallas as pl
from jax.experimental.pallas import tpu as pltpu
from jax.experimental.pallas import tpu_sc as plsc

N = 10000
E = 320000
D = 128
T = 4
STEPS = 8
B = 10
NPG = 1000

NC = 2
NS = 16
NW = NC * NS
EPW = E // NW
K = 80
NCH = EPW // K
NP_ = 10240
RPS = NP_ // NS


@functools.lru_cache(maxsize=None)
def _get_sc_aggregate():
    mesh = plsc.VectorSubcoreMesh(core_axis_name="c", subcore_axis_name="s")

    @functools.partial(
        pl.kernel,
        mesh=mesh,
        out_type=jax.ShapeDtypeStruct((NC, NP_, D), jnp.float32),
        scratch_types=[
            pltpu.VMEM((NCH, K), jnp.int32),
            pltpu.VMEM((NCH, K), jnp.int32),
            pltpu.VMEM((K, D), jnp.float32),
            pltpu.VMEM_SHARED((NP_, D), jnp.float32),
            pltpu.SemaphoreType.DMA,
        ],
    )
    def _sc_aggregate(table, gidxr, dstr, zeros, out, gidx_v, dst_v, rows, accum, sem):
        s = lax.axis_index("s")
        c = lax.axis_index("c")
        wid = s * NC + c

        pltpu.sync_copy(gidxr.at[wid], gidx_v)
        pltpu.sync_copy(dstr.at[wid], dst_v)
        pltpu.sync_copy(zeros.at[pl.ds(s * RPS, RPS)], accum.at[pl.ds(s * RPS, RPS)])
        plsc.subcore_barrier()

        def body(j, carry):
            pltpu.async_copy(table.at[gidx_v.at[j]], rows, sem).wait()
            pltpu.sync_copy(rows, accum.at[dst_v.at[j]], add=True)
            return carry

        lax.fori_loop(0, NCH, body, 0)

        plsc.subcore_barrier()
        pltpu.sync_copy(accum.at[pl.ds(s * RPS, RPS)],
                        out.at[c, pl.ds(s * RPS, RPS)])

    return _sc_aggregate


_BN = 2000
_NB = N // _BN


def _transform_body(h_ref, w_ref, b_ref, out_ref):
    h = h_ref[...]
    w = w_ref[0]
    out_ref[0] = lax.dot_general(h, w, (((1,), (1,)), ((), ())),
                                 preferred_element_type=jnp.float32) + b_ref[0]


def _tc_transform(h, W_et, b_et):
    return pl.pallas_call(
        _transform_body,
        grid=(_NB, T),
        in_specs=[
            pl.BlockSpec((_BN, D), lambda i, t: (i, 0)),
            pl.BlockSpec((1, D, D), lambda i, t: (t, 0, 0)),
            pl.BlockSpec((1, 1, D), lambda i, t: (t, 0, 0)),
        ],
        out_specs=pl.BlockSpec((1, _BN, D), lambda i, t: (t, i, 0)),
        out_shape=jax.ShapeDtypeStruct((T, N, D), jnp.float32),
    )(h, W_et, b_et)


def _gru(p0_ref, p1_ref, h_ref, wih_ref, whh_ref, bih_ref, bhh_ref):
    a = p0_ref[0] + p1_ref[0]
    h = h_ref[...]
    gi = lax.dot_general(a, wih_ref[...], (((1,), (1,)), ((), ())),
                         preferred_element_type=jnp.float32) + bih_ref[...]
    gh = lax.dot_general(h, whh_ref[...], (((1,), (1,)), ((), ())),
                         preferred_element_type=jnp.float32) + bhh_ref[...]
    r = jax.nn.sigmoid(gi[:, :D] + gh[:, :D])
    z = jax.nn.sigmoid(gi[:, D:2 * D] + gh[:, D:2 * D])
    n = jnp.tanh(gi[:, 2 * D:] + r * gh[:, 2 * D:])
    return (1.0 - z) * n + z * h


def _gru_transform_body(p0_ref, p1_ref, h_ref, wih_ref, whh_ref, bih_ref,
                        bhh_ref, wet_ref, bet_ref, hout_ref, tout_ref):
    hn = _gru(p0_ref, p1_ref, h_ref, wih_ref, whh_ref, bih_ref, bhh_ref)
    hout_ref[...] = hn
    for t in range(T):
        tout_ref[t] = lax.dot_general(hn, wet_ref[t], (((1,), (1,)), ((), ())),
                                      preferred_element_type=jnp.float32) \
            + bet_ref[t]


def _tc_gru_transform(parts, h, W_ih, W_hh, b_ih, b_hh, W_et, b_et):
    return pl.pallas_call(
        _gru_transform_body,
        grid=(_NB,),
        in_specs=[
            pl.BlockSpec((1, _BN, D), lambda i: (0, i, 0)),
            pl.BlockSpec((1, _BN, D), lambda i: (1, i, 0)),
            pl.BlockSpec((_BN, D), lambda i: (i, 0)),
            pl.BlockSpec((3 * D, D), lambda i: (0, 0)),
            pl.BlockSpec((3 * D, D), lambda i: (0, 0)),
            pl.BlockSpec((1, 3 * D), lambda i: (0, 0)),
            pl.BlockSpec((1, 3 * D), lambda i: (0, 0)),
            pl.BlockSpec((T, D, D), lambda i: (0, 0, 0)),
            pl.BlockSpec((T, 1, D), lambda i: (0, 0, 0)),
        ],
        out_specs=[
            pl.BlockSpec((_BN, D), lambda i: (i, 0)),
            pl.BlockSpec((T, _BN, D), lambda i: (0, i, 0)),
        ],
        out_shape=[
            jax.ShapeDtypeStruct((N, D), jnp.float32),
            jax.ShapeDtypeStruct((T, N, D), jnp.float32),
        ],
    )(parts, parts, h, W_ih, W_hh, b_ih, b_hh, W_et, b_et)


def _gru_pool_body(p0_ref, p1_ref, h_ref, wih_ref, whh_ref, bih_ref, bhh_ref,
                   wcls_ref, bcls_ref, out_ref):
    hn = _gru(p0_ref, p1_ref, h_ref, wih_ref, whh_ref, bih_ref, bhh_ref)
    w = wcls_ref[...]
    b = bcls_ref[...]
    g0 = jnp.sum(hn[:NPG], axis=0, keepdims=True)
    g1 = jnp.sum(hn[NPG:], axis=0, keepdims=True)
    out_ref[0] = jax.nn.sigmoid(jnp.sum(g0 * w, axis=1, keepdims=True) + b)
    out_ref[1] = jax.nn.sigmoid(jnp.sum(g1 * w, axis=1, keepdims=True) + b)


def _tc_gru_pool(parts, h, W_ih, W_hh, b_ih, b_hh, W_cls, b_cls):
    return pl.pallas_call(
        _gru_pool_body,
        grid=(_NB,),
        in_specs=[
            pl.BlockSpec((1, _BN, D), lambda i: (0, i, 0)),
            pl.BlockSpec((1, _BN, D), lambda i: (1, i, 0)),
            pl.BlockSpec((_BN, D), lambda i: (i, 0)),
            pl.BlockSpec((3 * D, D), lambda i: (0, 0)),
            pl.BlockSpec((3 * D, D), lambda i: (0, 0)),
            pl.BlockSpec((1, 3 * D), lambda i: (0, 0)),
            pl.BlockSpec((1, 3 * D), lambda i: (0, 0)),
            pl.BlockSpec((1, D), lambda i: (0, 0)),
            pl.BlockSpec((1, D), lambda i: (0, 0)),
        ],
        out_specs=pl.BlockSpec((2, 1, D), lambda i: (i, 0, 0)),
        out_shape=jax.ShapeDtypeStruct((B, 1, D), jnp.float32),
    )(parts, parts, h, W_ih, W_hh, b_ih, b_hh, W_cls, b_cls)


def kernel(features, edge_index, edge_types, W_et, b_et, W_ih, W_hh, b_ih, b_hh, W_cls, b_cls):
    src = edge_index[0]
    dst = edge_index[1]
    gidx = (edge_types * N + src).reshape(NW, NCH, K)
    dst3 = dst.reshape(NW, NCH, K)
    zeros = jnp.zeros((NP_, D), jnp.float32)
    b_et3 = b_et.reshape(T, 1, D)
    b_ih2 = b_ih.reshape(1, 3 * D)
    b_hh2 = b_hh.reshape(1, 3 * D)
    b_cls2 = jnp.broadcast_to(b_cls.reshape(1, 1), (1, D))

    agg = _get_sc_aggregate()
    h = features
    table = _tc_transform(h, W_et, b_et3).reshape(T * N, D)
    parts = agg(table, gidx, dst3, zeros)
    for _ in range(STEPS - 1):
        h, table = _tc_gru_transform(parts, h, W_ih, W_hh, b_ih2, b_hh2,
                                     W_et, b_et3)
        parts = agg(table.reshape(T * N, D), gidx, dst3, zeros)
    out = _tc_gru_pool(parts, h, W_ih, W_hh, b_ih2, b_hh2, W_cls, b_cls2)
    return out[:, 0, 0]


# speedup vs baseline: 1.7353x; 1.1568x over previous
import functools

import jax
import jax.numpy as jnp
from jax import lax
from jax.experimental import pallas as pl
from jax.experimental.pallas import tpu as pltpu
from jax.experimental.pallas import tpu_sc as plsc

N = 10000
E = 320000
D = 128
T = 4
STEPS = 8
B = 10
NPG = 1000

NC = 2
NS = 16
NW = NC * NS
EPW = E // NW
K = 128
NCH = (EPW + K - 1) // K
EPP = NCH * K
NP_ = 10112
RPS = NP_ // NS


@functools.lru_cache(maxsize=None)
def _get_sc_aggregate():
    mesh = plsc.VectorSubcoreMesh(core_axis_name="c", subcore_axis_name="s")

    @functools.partial(
        pl.kernel,
        mesh=mesh,
        out_type=jax.ShapeDtypeStruct((NC, NP_, D), jnp.float32),
        scratch_types=[
            pltpu.VMEM((NCH, K), jnp.int32),
            pltpu.VMEM((NCH, K), jnp.int32),
            pltpu.VMEM((K, D), jnp.float32),
            pltpu.VMEM_SHARED((NP_, D), jnp.float32),
            pltpu.SemaphoreType.DMA,
        ],
    )
    def _sc_aggregate(table, gidxr, dstr, zeros, out, gidx_v, dst_v, rows, accum, sem):
        s = lax.axis_index("s")
        c = lax.axis_index("c")
        wid = s * NC + c

        pltpu.sync_copy(gidxr.at[wid], gidx_v)
        pltpu.sync_copy(dstr.at[wid], dst_v)
        pltpu.sync_copy(zeros.at[pl.ds(s * RPS, RPS)], accum.at[pl.ds(s * RPS, RPS)])
        plsc.subcore_barrier()

        def body(j, carry):
            pltpu.async_copy(table.at[gidx_v.at[j]], rows, sem).wait()
            pltpu.sync_copy(rows, accum.at[dst_v.at[j]], add=True)
            return carry

        lax.fori_loop(0, NCH, body, 0)

        plsc.subcore_barrier()
        pltpu.sync_copy(accum.at[pl.ds(s * RPS, RPS)],
                        out.at[c, pl.ds(s * RPS, RPS)])

    return _sc_aggregate


_BN = 2000
_NB = N // _BN


def _transform_body(h_ref, w_ref, b_ref, out_ref):
    h = h_ref[...]
    w = w_ref[0]
    out_ref[0] = lax.dot_general(h, w, (((1,), (1,)), ((), ())),
                                 preferred_element_type=jnp.float32) + b_ref[0]


def _tc_transform(h, W_et, b_et):
    return pl.pallas_call(
        _transform_body,
        grid=(_NB, T),
        in_specs=[
            pl.BlockSpec((_BN, D), lambda i, t: (i, 0)),
            pl.BlockSpec((1, D, D), lambda i, t: (t, 0, 0)),
            pl.BlockSpec((1, 1, D), lambda i, t: (t, 0, 0)),
        ],
        out_specs=pl.BlockSpec((1, _BN, D), lambda i, t: (t, i, 0)),
        out_shape=jax.ShapeDtypeStruct((T, N, D), jnp.float32),
    )(h, W_et, b_et)


def _gru(p0_ref, p1_ref, h_ref, wih_ref, whh_ref, bih_ref, bhh_ref):
    a = p0_ref[0] + p1_ref[0]
    h = h_ref[...]
    gi = lax.dot_general(a, wih_ref[...], (((1,), (1,)), ((), ())),
                         preferred_element_type=jnp.float32) + bih_ref[...]
    gh = lax.dot_general(h, whh_ref[...], (((1,), (1,)), ((), ())),
                         preferred_element_type=jnp.float32) + bhh_ref[...]
    r = jax.nn.sigmoid(gi[:, :D] + gh[:, :D])
    z = jax.nn.sigmoid(gi[:, D:2 * D] + gh[:, D:2 * D])
    n = jnp.tanh(gi[:, 2 * D:] + r * gh[:, 2 * D:])
    return (1.0 - z) * n + z * h


def _gru_transform_body(p0_ref, p1_ref, h_ref, wih_ref, whh_ref, bih_ref,
                        bhh_ref, wet_ref, bet_ref, hout_ref, tout_ref):
    hn = _gru(p0_ref, p1_ref, h_ref, wih_ref, whh_ref, bih_ref, bhh_ref)
    hout_ref[...] = hn
    for t in range(T):
        tout_ref[t] = lax.dot_general(hn, wet_ref[t], (((1,), (1,)), ((), ())),
                                      preferred_element_type=jnp.float32) \
            + bet_ref[t]


def _tc_gru_transform(parts, h, W_ih, W_hh, b_ih, b_hh, W_et, b_et):
    return pl.pallas_call(
        _gru_transform_body,
        grid=(_NB,),
        in_specs=[
            pl.BlockSpec((1, _BN, D), lambda i: (0, i, 0)),
            pl.BlockSpec((1, _BN, D), lambda i: (1, i, 0)),
            pl.BlockSpec((_BN, D), lambda i: (i, 0)),
            pl.BlockSpec((3 * D, D), lambda i: (0, 0)),
            pl.BlockSpec((3 * D, D), lambda i: (0, 0)),
            pl.BlockSpec((1, 3 * D), lambda i: (0, 0)),
            pl.BlockSpec((1, 3 * D), lambda i: (0, 0)),
            pl.BlockSpec((T, D, D), lambda i: (0, 0, 0)),
            pl.BlockSpec((T, 1, D), lambda i: (0, 0, 0)),
        ],
        out_specs=[
            pl.BlockSpec((_BN, D), lambda i: (i, 0)),
            pl.BlockSpec((T, _BN, D), lambda i: (0, i, 0)),
        ],
        out_shape=[
            jax.ShapeDtypeStruct((N, D), jnp.float32),
            jax.ShapeDtypeStruct((T, N, D), jnp.float32),
        ],
    )(parts, parts, h, W_ih, W_hh, b_ih, b_hh, W_et, b_et)


def _gru_pool_body(p0_ref, p1_ref, h_ref, wih_ref, whh_ref, bih_ref, bhh_ref,
                   wcls_ref, bcls_ref, out_ref):
    hn = _gru(p0_ref, p1_ref, h_ref, wih_ref, whh_ref, bih_ref, bhh_ref)
    w = wcls_ref[...]
    b = bcls_ref[...]
    g0 = jnp.sum(hn[:NPG], axis=0, keepdims=True)
    g1 = jnp.sum(hn[NPG:], axis=0, keepdims=True)
    out_ref[0] = jax.nn.sigmoid(jnp.sum(g0 * w, axis=1, keepdims=True) + b)
    out_ref[1] = jax.nn.sigmoid(jnp.sum(g1 * w, axis=1, keepdims=True) + b)


def _tc_gru_pool(parts, h, W_ih, W_hh, b_ih, b_hh, W_cls, b_cls):
    return pl.pallas_call(
        _gru_pool_body,
        grid=(_NB,),
        in_specs=[
            pl.BlockSpec((1, _BN, D), lambda i: (0, i, 0)),
            pl.BlockSpec((1, _BN, D), lambda i: (1, i, 0)),
            pl.BlockSpec((_BN, D), lambda i: (i, 0)),
            pl.BlockSpec((3 * D, D), lambda i: (0, 0)),
            pl.BlockSpec((3 * D, D), lambda i: (0, 0)),
            pl.BlockSpec((1, 3 * D), lambda i: (0, 0)),
            pl.BlockSpec((1, 3 * D), lambda i: (0, 0)),
            pl.BlockSpec((1, D), lambda i: (0, 0)),
            pl.BlockSpec((1, D), lambda i: (0, 0)),
        ],
        out_specs=pl.BlockSpec((2, 1, D), lambda i: (i, 0, 0)),
        out_shape=jax.ShapeDtypeStruct((B, 1, D), jnp.float32),
    )(parts, parts, h, W_ih, W_hh, b_ih, b_hh, W_cls, b_cls)


def kernel(features, edge_index, edge_types, W_et, b_et, W_ih, W_hh, b_ih, b_hh, W_cls, b_cls):
    src = edge_index[0]
    dst = edge_index[1]
    pad = EPP - EPW
    padg = jnp.arange(pad, dtype=jnp.int32)
    gidx = jnp.concatenate(
        [(edge_types * N + src).reshape(NW, EPW),
         jnp.broadcast_to(padg, (NW, pad))], axis=1).reshape(NW, NCH, K)
    dst3 = jnp.concatenate(
        [dst.reshape(NW, EPW),
         jnp.broadcast_to(N + padg, (NW, pad))], axis=1).reshape(NW, NCH, K)
    b_et3 = b_et.reshape(T, 1, D)
    b_ih2 = b_ih.reshape(1, 3 * D)
    b_hh2 = b_hh.reshape(1, 3 * D)
    b_cls2 = jnp.broadcast_to(b_cls.reshape(1, 1), (1, D))

    zeros = jnp.zeros((NP_, D), jnp.float32)
    agg = _get_sc_aggregate()
    h = features
    table = _tc_transform(h, W_et, b_et3).reshape(T * N, D)
    parts = agg(table, gidx, dst3, zeros)
    for _ in range(STEPS - 1):
        h, table = _tc_gru_transform(parts, h, W_ih, W_hh, b_ih2, b_hh2,
                                     W_et, b_et3)
        parts = agg(table.reshape(T * N, D), gidx, dst3, zeros)
    out = _tc_gru_pool(parts, h, W_ih, W_hh, b_ih2, b_hh2, W_cls, b_cls2)
    return out[:, 0, 0]


# speedup vs baseline: 1.7404x; 1.0030x over previous
import functools

import jax
import jax.numpy as jnp
from jax import lax
from jax.experimental import pallas as pl
from jax.experimental.pallas import tpu as pltpu
from jax.experimental.pallas import tpu_sc as plsc

N = 10000
E = 320000
D = 128
T = 4
STEPS = 8
B = 10
NPG = 1000

NC = 2
NS = 16
NW = NC * NS
EPW = E // NW
K = 128
NCH = (EPW + K - 1) // K
EPP = NCH * K
NP_ = 10112
RPS = NP_ // NS


@functools.lru_cache(maxsize=None)
def _get_sc_aggregate():
    mesh = plsc.VectorSubcoreMesh(core_axis_name="c", subcore_axis_name="s")

    @functools.partial(
        pl.kernel,
        mesh=mesh,
        out_type=jax.ShapeDtypeStruct((NC, NP_, D), jnp.float32),
        scratch_types=[
            pltpu.VMEM((NCH, K), jnp.int32),
            pltpu.VMEM((NCH, K), jnp.int32),
            pltpu.VMEM((K, D), jnp.float32),
            pltpu.VMEM_SHARED((NP_, D), jnp.float32),
            pltpu.SemaphoreType.DMA,
        ],
    )
    def _sc_aggregate(table, gidxr, dstr, out, gidx_v, dst_v, rows, accum, sem):
        s = lax.axis_index("s")
        c = lax.axis_index("c")
        wid = s * NC + c

        pltpu.sync_copy(gidxr.at[wid], gidx_v)
        pltpu.sync_copy(dstr.at[wid], dst_v)

        z16 = jnp.zeros((16,), jnp.float32)

        def zstore(i, carry):
            rows[i // 8, pl.ds(lax.rem(i, 8) * 16, 16)] = z16
            return carry

        lax.fori_loop(0, K * 8, zstore, 0)

        def zcopy(q, carry):
            pltpu.sync_copy(rows, accum.at[pl.ds(s * RPS + q * K, K)])
            return carry

        lax.fori_loop(0, RPS // K, zcopy, 0)
        plsc.subcore_barrier()

        def body(j, carry):
            pltpu.async_copy(table.at[gidx_v.at[j]], rows, sem).wait()
            pltpu.sync_copy(rows, accum.at[dst_v.at[j]], add=True)
            return carry

        lax.fori_loop(0, NCH, body, 0)

        plsc.subcore_barrier()
        pltpu.sync_copy(accum.at[pl.ds(s * RPS, RPS)],
                        out.at[c, pl.ds(s * RPS, RPS)])

    return _sc_aggregate


_BN = 2000
_NB = N // _BN


def _transform_body(h_ref, w_ref, b_ref, out_ref):
    h = h_ref[...]
    w = w_ref[0]
    out_ref[0] = lax.dot_general(h, w, (((1,), (1,)), ((), ())),
                                 preferred_element_type=jnp.float32) + b_ref[0]


def _tc_transform(h, W_et, b_et):
    return pl.pallas_call(
        _transform_body,
        grid=(_NB, T),
        in_specs=[
            pl.BlockSpec((_BN, D), lambda i, t: (i, 0)),
            pl.BlockSpec((1, D, D), lambda i, t: (t, 0, 0)),
            pl.BlockSpec((1, 1, D), lambda i, t: (t, 0, 0)),
        ],
        out_specs=pl.BlockSpec((1, _BN, D), lambda i, t: (t, i, 0)),
        out_shape=jax.ShapeDtypeStruct((T, N, D), jnp.float32),
    )(h, W_et, b_et)


def _gru(p0_ref, p1_ref, h_ref, wih_ref, whh_ref, bih_ref, bhh_ref):
    a = p0_ref[0] + p1_ref[0]
    h = h_ref[...]
    gi = lax.dot_general(a, wih_ref[...], (((1,), (1,)), ((), ())),
                         preferred_element_type=jnp.float32) + bih_ref[...]
    gh = lax.dot_general(h, whh_ref[...], (((1,), (1,)), ((), ())),
                         preferred_element_type=jnp.float32) + bhh_ref[...]
    r = jax.nn.sigmoid(gi[:, :D] + gh[:, :D])
    z = jax.nn.sigmoid(gi[:, D:2 * D] + gh[:, D:2 * D])
    n = jnp.tanh(gi[:, 2 * D:] + r * gh[:, 2 * D:])
    return (1.0 - z) * n + z * h


def _gru_transform_body(p0_ref, p1_ref, h_ref, wih_ref, whh_ref, bih_ref,
                        bhh_ref, wet_ref, bet_ref, hout_ref, tout_ref):
    hn = _gru(p0_ref, p1_ref, h_ref, wih_ref, whh_ref, bih_ref, bhh_ref)
    hout_ref[...] = hn
    for t in range(T):
        tout_ref[t] = lax.dot_general(hn, wet_ref[t], (((1,), (1,)), ((), ())),
                                      preferred_element_type=jnp.float32) \
            + bet_ref[t]


def _tc_gru_transform(parts, h, W_ih, W_hh, b_ih, b_hh, W_et, b_et):
    return pl.pallas_call(
        _gru_transform_body,
        grid=(_NB,),
        in_specs=[
            pl.BlockSpec((1, _BN, D), lambda i: (0, i, 0)),
            pl.BlockSpec((1, _BN, D), lambda i: (1, i, 0)),
            pl.BlockSpec((_BN, D), lambda i: (i, 0)),
            pl.BlockSpec((3 * D, D), lambda i: (0, 0)),
            pl.BlockSpec((3 * D, D), lambda i: (0, 0)),
            pl.BlockSpec((1, 3 * D), lambda i: (0, 0)),
            pl.BlockSpec((1, 3 * D), lambda i: (0, 0)),
            pl.BlockSpec((T, D, D), lambda i: (0, 0, 0)),
            pl.BlockSpec((T, 1, D), lambda i: (0, 0, 0)),
        ],
        out_specs=[
            pl.BlockSpec((_BN, D), lambda i: (i, 0)),
            pl.BlockSpec((T, _BN, D), lambda i: (0, i, 0)),
        ],
        out_shape=[
            jax.ShapeDtypeStruct((N, D), jnp.float32),
            jax.ShapeDtypeStruct((T, N, D), jnp.float32),
        ],
    )(parts, parts, h, W_ih, W_hh, b_ih, b_hh, W_et, b_et)


def _gru_pool_body(p0_ref, p1_ref, h_ref, wih_ref, whh_ref, bih_ref, bhh_ref,
                   wcls_ref, bcls_ref, out_ref):
    hn = _gru(p0_ref, p1_ref, h_ref, wih_ref, whh_ref, bih_ref, bhh_ref)
    w = wcls_ref[...]
    b = bcls_ref[...]
    g0 = jnp.sum(hn[:NPG], axis=0, keepdims=True)
    g1 = jnp.sum(hn[NPG:], axis=0, keepdims=True)
    out_ref[0] = jax.nn.sigmoid(jnp.sum(g0 * w, axis=1, keepdims=True) + b)
    out_ref[1] = jax.nn.sigmoid(jnp.sum(g1 * w, axis=1, keepdims=True) + b)


def _tc_gru_pool(parts, h, W_ih, W_hh, b_ih, b_hh, W_cls, b_cls):
    return pl.pallas_call(
        _gru_pool_body,
        grid=(_NB,),
        in_specs=[
            pl.BlockSpec((1, _BN, D), lambda i: (0, i, 0)),
            pl.BlockSpec((1, _BN, D), lambda i: (1, i, 0)),
            pl.BlockSpec((_BN, D), lambda i: (i, 0)),
            pl.BlockSpec((3 * D, D), lambda i: (0, 0)),
            pl.BlockSpec((3 * D, D), lambda i: (0, 0)),
            pl.BlockSpec((1, 3 * D), lambda i: (0, 0)),
            pl.BlockSpec((1, 3 * D), lambda i: (0, 0)),
            pl.BlockSpec((1, D), lambda i: (0, 0)),
            pl.BlockSpec((1, D), lambda i: (0, 0)),
        ],
        out_specs=pl.BlockSpec((2, 1, D), lambda i: (i, 0, 0)),
        out_shape=jax.ShapeDtypeStruct((B, 1, D), jnp.float32),
    )(parts, parts, h, W_ih, W_hh, b_ih, b_hh, W_cls, b_cls)


def kernel(features, edge_index, edge_types, W_et, b_et, W_ih, W_hh, b_ih, b_hh, W_cls, b_cls):
    src = edge_index[0]
    dst = edge_index[1]
    pad = EPP - EPW
    padg = jnp.arange(pad, dtype=jnp.int32)
    gidx = jnp.concatenate(
        [(edge_types * N + src).reshape(NW, EPW),
         jnp.broadcast_to(padg, (NW, pad))], axis=1).reshape(NW, NCH, K)
    dst3 = jnp.concatenate(
        [dst.reshape(NW, EPW),
         jnp.broadcast_to(N + padg, (NW, pad))], axis=1).reshape(NW, NCH, K)
    b_et3 = b_et.reshape(T, 1, D)
    b_ih2 = b_ih.reshape(1, 3 * D)
    b_hh2 = b_hh.reshape(1, 3 * D)
    b_cls2 = jnp.broadcast_to(b_cls.reshape(1, 1), (1, D))

    agg = _get_sc_aggregate()
    h = features
    table = _tc_transform(h, W_et, b_et3).reshape(T * N, D)
    parts = agg(table, gidx, dst3)
    for _ in range(STEPS - 1):
        h, table = _tc_gru_transform(parts, h, W_ih, W_hh, b_ih2, b_hh2,
                                     W_et, b_et3)
        parts = agg(table.reshape(T * N, D), gidx, dst3)
    out = _tc_gru_pool(parts, h, W_ih, W_hh, b_ih2, b_hh2, W_cls, b_cls2)
    return out[:, 0, 0]
